# baseline (speedup 1.0000x reference)
import functools

import jax
import jax.numpy as jnp
from jax import lax
from jax.experimental import pallas as pl
from jax.experimental.pallas import tpu as pltpu
from jax.experimental.pallas import tpu_sc as plsc

VOCAB = 1000000
D_BOT = 64
D_MODEL = 128
B = 4096
L = 200
N = B * L
NCHUNK = N // 128

_NC = 2
_NS = 16
_NW = _NC * _NS

_CHUNKS_W = NCHUNK // _NW
_K = 2
_ITERS = _CHUNKS_W // _K

_BLKV = 8192
_NVBLK = -(-VOCAB // _BLKV)
_VPAD = _NVBLK * _BLKV


def _proj_body(t_ref, w_ref, b_ref, o_ref):
    o_ref[...] = (
        jax.lax.dot_general(
            t_ref[...].astype(jnp.bfloat16),
            w_ref[...].astype(jnp.bfloat16),
            dimension_numbers=(((0,), (0,)), ((), ())),
            preferred_element_type=jnp.float32,
        )
        + b_ref[...]
    )


def _tc_project_table(tableT, W, b1):
    return pl.pallas_call(
        _proj_body,
        grid=(_NVBLK,),
        in_specs=[
            pl.BlockSpec((D_BOT, _BLKV), lambda i: (0, i)),
            pl.BlockSpec((D_BOT, D_MODEL), lambda i: (0, 0)),
            pl.BlockSpec((1, D_MODEL), lambda i: (0, 0)),
        ],
        out_specs=pl.BlockSpec((_BLKV, D_MODEL), lambda i: (i, 0)),
        out_shape=jax.ShapeDtypeStruct((_VPAD, D_MODEL), jnp.float32),
        compiler_params=pltpu.CompilerParams(
            dimension_semantics=("arbitrary",),
        ),
    )(tableT, W, b1)


def _sc_gather(x2d, tableWb):
    mesh = plsc.VectorSubcoreMesh(core_axis_name="c", subcore_axis_name="s")

    @functools.partial(
        pl.kernel,
        mesh=mesh,
        out_type=jax.ShapeDtypeStruct((NCHUNK, 128, D_MODEL), jnp.float32),
        scratch_types=[
            pltpu.VMEM((2, _K, 128), jnp.int32),
            pltpu.VMEM((2, _K, 128, D_MODEL), jnp.float32),
            pltpu.SemaphoreType.DMA,
            pltpu.SemaphoreType.DMA,
        ],
        compiler_params=pltpu.CompilerParams(use_tc_tiling_on_sc=True),
    )
    def gather_kernel(idx_hbm, table_hbm, out_hbm, idx_v, rows_v, sem, sem_s):
        wid = lax.axis_index("s") * _NC + lax.axis_index("c")
        base = wid * _CHUNKS_W
        half = _ITERS // 2

        def fire(i, bb):
            c0 = base + i * _K
            pltpu.sync_copy(idx_hbm.at[pl.ds(c0, _K)], idx_v.at[bb])
            for j in range(_K):
                pltpu.async_copy(
                    table_hbm.at[idx_v.at[bb, j]], rows_v.at[bb, j], sem
                )

        def wait_gathers(bb):
            for j in range(_K):
                pltpu.make_async_copy(
                    table_hbm.at[idx_v.at[bb, j]], rows_v.at[bb, j], sem
                ).wait()

        def start_store(i, bb):
            c0 = base + i * _K
            pltpu.async_copy(rows_v.at[bb], out_hbm.at[pl.ds(c0, _K)], sem_s)

        def wait_store(bb):
            pltpu.make_async_copy(
                rows_v.at[bb], out_hbm.at[pl.ds(base, _K)], sem_s
            ).wait()

        fire(0, 0)

        def body(g, carry):
            i0 = 2 * g

            @pl.when(g > 0)
            def _():
                wait_store(1)

            fire(i0 + 1, 1)
            wait_gathers(0)
            start_store(i0, 0)

            @pl.when(g < half - 1)
            def _():
                wait_store(0)
                fire(i0 + 2, 0)

            wait_gathers(1)
            start_store(i0 + 1, 1)
            return carry

        lax.fori_loop(0, half, body, 0)
        wait_store(0)
        wait_store(1)

    return gather_kernel(x2d, tableWb)


def kernel(x, table, W, b):
    x2d = x.astype(jnp.int32).reshape(NCHUNK, 128)
    tableWb = _tc_project_table(table.T, W, b.reshape(1, D_MODEL))
    out = _sc_gather(x2d, tableWb)
    return out.reshape(B, L, D_MODEL)


# speedup vs baseline: 1.0231x; 1.0231x over previous
import functools

import jax
import jax.numpy as jnp
from jax import lax
from jax.experimental import pallas as pl
from jax.experimental.pallas import tpu as pltpu
from jax.experimental.pallas import tpu_sc as plsc

VOCAB = 1000000
D_BOT = 64
D_MODEL = 128
B = 4096
L = 200
N = B * L
NCHUNK = N // 128

_NC = 2
_NS = 16
_NW = _NC * _NS

_CHUNKS_W = NCHUNK // _NW
_K = 2
_ITERS = _CHUNKS_W // _K

_BLKV = 16384
_NVBLK = -(-VOCAB // _BLKV)
_VPAD = _NVBLK * _BLKV


def _proj_body(t_ref, w_ref, b_ref, o_ref):
    o_ref[...] = (
        jax.lax.dot_general(
            t_ref[...].astype(jnp.bfloat16),
            w_ref[...].astype(jnp.bfloat16),
            dimension_numbers=(((0,), (0,)), ((), ())),
            preferred_element_type=jnp.float32,
        )
        + b_ref[...]
    )


def _tc_project_table(tableT, W, b1):
    return pl.pallas_call(
        _proj_body,
        grid=(_NVBLK,),
        in_specs=[
            pl.BlockSpec((D_BOT, _BLKV), lambda i: (0, i)),
            pl.BlockSpec((D_BOT, D_MODEL), lambda i: (0, 0)),
            pl.BlockSpec((1, D_MODEL), lambda i: (0, 0)),
        ],
        out_specs=pl.BlockSpec((_BLKV, D_MODEL), lambda i: (i, 0)),
        out_shape=jax.ShapeDtypeStruct((_VPAD, D_MODEL), jnp.float32),
        compiler_params=pltpu.CompilerParams(
            dimension_semantics=("arbitrary",),
        ),
    )(tableT, W, b1)


def _sc_gather(x2d, tableWb):
    mesh = plsc.VectorSubcoreMesh(core_axis_name="c", subcore_axis_name="s")

    @functools.partial(
        pl.kernel,
        mesh=mesh,
        out_type=jax.ShapeDtypeStruct((NCHUNK, 128, D_MODEL), jnp.float32),
        scratch_types=[
            pltpu.VMEM((2, _K, 128), jnp.int32),
            pltpu.VMEM((2, _K, 128, D_MODEL), jnp.float32),
            pltpu.SemaphoreType.DMA,
            pltpu.SemaphoreType.DMA,
        ],
        compiler_params=pltpu.CompilerParams(use_tc_tiling_on_sc=True),
    )
    def gather_kernel(idx_hbm, table_hbm, out_hbm, idx_v, rows_v, sem, sem_s):
        wid = lax.axis_index("s") * _NC + lax.axis_index("c")
        base = wid * _CHUNKS_W
        half = _ITERS // 2

        def fire(i, bb):
            c0 = base + i * _K
            pltpu.sync_copy(idx_hbm.at[pl.ds(c0, _K)], idx_v.at[bb])
            for j in range(_K):
                pltpu.async_copy(
                    table_hbm.at[idx_v.at[bb, j]], rows_v.at[bb, j], sem
                )

        def wait_gathers(bb):
            for j in range(_K):
                pltpu.make_async_copy(
                    table_hbm.at[idx_v.at[bb, j]], rows_v.at[bb, j], sem
                ).wait()

        def start_store(i, bb):
            c0 = base + i * _K
            pltpu.async_copy(rows_v.at[bb], out_hbm.at[pl.ds(c0, _K)], sem_s)

        def wait_store(bb):
            pltpu.make_async_copy(
                rows_v.at[bb], out_hbm.at[pl.ds(base, _K)], sem_s
            ).wait()

        fire(0, 0)

        def body(g, carry):
            i0 = 2 * g

            @pl.when(g > 0)
            def _():
                wait_store(1)

            fire(i0 + 1, 1)
            wait_gathers(0)
            start_store(i0, 0)

            @pl.when(g < half - 1)
            def _():
                wait_store(0)
                fire(i0 + 2, 0)

            wait_gathers(1)
            start_store(i0 + 1, 1)
            return carry

        lax.fori_loop(0, half, body, 0)
        wait_store(0)
        wait_store(1)

    return gather_kernel(x2d, tableWb)


def kernel(x, table, W, b):
    x2d = x.astype(jnp.int32).reshape(NCHUNK, 128)
    tableWb = _tc_project_table(table.T, W, b.reshape(1, D_MODEL))
    out = _sc_gather(x2d, tableWb)
    return out.reshape(B, L, D_MODEL)
